# Initial kernel scaffold; baseline (speedup 1.0000x reference)
import jax
import jax.numpy as jnp
from jax.experimental import pallas as pl


def kernel(x_in, edge_attr, params, edge_index):
    raise NotImplementedError("write your pallas kernel here")


# speedup vs baseline: 3.6772x; 3.6772x over previous
import functools

import jax
import jax.numpy as jnp
from jax import lax
from jax.experimental import pallas as pl
from jax.experimental.pallas import tpu as pltpu
from jax.experimental.pallas import tpu_sc as plsc

N = 10000
NP = 10240
E = 320000
INDIM = 128
EDGE_DIM = 16

CH = 128
NCHUNK = E // CH
NC = 2
NS = 16
NW = NC * NS
JMAX = -(-NCHUNK // NW)
RPT = NP // NS

EB = 2560
NB = 1280


def _ln(x, g, b, eps=1e-6):
    mu = jnp.mean(x, axis=-1, keepdims=True)
    var = jnp.mean((x - mu) ** 2, axis=-1, keepdims=True)
    return (x - mu) / jnp.sqrt(var + eps) * g + b


def _full(a):
    return pl.BlockSpec(a.shape, lambda i: (0,) * a.ndim)


def _blk(shape):
    nd = len(shape)
    return pl.BlockSpec(shape, lambda i: (i,) + (0,) * (nd - 1))


def _edge_enc_body(ea, g, b, w1, b1, w2, b2, w3, b3, cw1, cb1, cw2, cb2, out):
    eav = ea[...]
    e = _ln(eav, g[...], b[...])
    e = jax.nn.relu(e @ w1[...] + b1[...])
    e = jax.nn.relu(e @ w2[...] + b2[...])
    e = e @ w3[...] + b3[...]
    h = jax.nn.relu(eav @ cw1[...] + cb1[...])
    ew = jax.nn.sigmoid(h @ cw2[...] + cb2[...])
    out[...] = e * ew


def _edge_enc(e_attr, p):
    ins = (p['ee_ln_g'], p['ee_ln_b'], p['ee_W1'], p['ee_b1'], p['ee_W2'],
           p['ee_b2'], p['ee_W3'], p['ee_b3'], p['ec_W1'], p['ec_b1'],
           p['ec_W2'], p['ec_b2'])
    return pl.pallas_call(
        _edge_enc_body,
        grid=(E // EB,),
        in_specs=[_blk((EB, EDGE_DIM))] + [_full(a) for a in ins],
        out_specs=_blk((EB, 128)),
        out_shape=jax.ShapeDtypeStruct((E, 128), jnp.float32),
    )(e_attr, *ins)


def _node_prep_body(xin, dummy, g, b, w1ds, x_out, p1_out):
    xv = xin[...]
    inv = xv[:, :1] == -999.0
    xv = jnp.where(inv, dummy[...][None, :], xv)
    xv = _ln(xv, g[...], b[...])
    x_out[...] = xv
    p1_out[...] = xv @ w1ds[...]


def _node_prep(x_pad, p):
    w1ds = jnp.concatenate([p['c1_W1'][:INDIM], p['c1_W1'][INDIM:2 * INDIM]],
                           axis=1)
    ins = (p['dummy'], p['bn0_g'], p['bn0_b'], w1ds)
    return pl.pallas_call(
        _node_prep_body,
        grid=(NP // NB,),
        in_specs=[_blk((NB, INDIM))] + [_full(a) for a in ins],
        out_specs=[_blk((NB, INDIM)), _blk((NB, 128))],
        out_shape=[jax.ShapeDtypeStruct((NP, INDIM), jnp.float32),
                   jax.ShapeDtypeStruct((NP, 128), jnp.float32)],
    )(x_pad, *ins)


def _conv_mlp(pre, e_enc, pre_cols, w1e, b1, w2, b2, w3, b3):
    def body(pre_r, e_r, w1e_r, b1_r, w2_r, b2_r, w3_r, b3_r, out):
        h = jax.nn.relu(pre_r[...][:, :pre_cols] + e_r[...] @ w1e_r[...]
                        + b1_r[...])
        h = jax.nn.relu(h @ w2_r[...] + b2_r[...])
        out[...] = h @ w3_r[...] + b3_r[...]

    ins = (w1e, b1, w2, b2, w3, b3)
    return pl.pallas_call(
        body,
        grid=(E // EB,),
        in_specs=[_blk((EB, 128)), _blk((EB, 128))] + [_full(a) for a in ins],
        out_specs=_blk((EB, 128)),
        out_shape=jax.ShapeDtypeStruct((E, 128), jnp.float32),
    )(pre, e_enc, *ins)


def _node_mid_body(sp, cntp, g, b, w2d, w2s, pd_out, ps_out, cnt_out):
    cv = cntp[...]
    cnt = cv[0, :, 0] + cv[1, :, 0]
    sv = sp[...][:, :, :64]
    s = sv[0] + sv[1]
    x1 = s / jnp.maximum(cnt, 1.0)[:, None]
    x1 = _ln(x1, g[...], b[...])
    x1 = jnp.where(x1 >= 0, x1, 0.01 * x1)
    pd_out[...] = x1 @ w2d[...]
    ps_out[...] = x1 @ w2s[...]
    cnt_out[...] = cnt[:, None]


def _node_mid(s_parts, cnt_parts, p):
    w2d = p['c2_W1'][:64]
    w2s = p['c2_W1'][64:128]
    ins = (p['bn1_g'], p['bn1_b'], w2d, w2s)
    return pl.pallas_call(
        _node_mid_body,
        grid=(NP // NB,),
        in_specs=[pl.BlockSpec((NC, NB, 128), lambda i: (0, i, 0)),
                  pl.BlockSpec((NC, NB, 128), lambda i: (0, i, 0))]
                 + [_full(a) for a in ins],
        out_specs=[_blk((NB, 128)), _blk((NB, 128)), _blk((NB, 1))],
        out_shape=[jax.ShapeDtypeStruct((NP, 128), jnp.float32),
                   jax.ShapeDtypeStruct((NP, 128), jnp.float32),
                   jax.ShapeDtypeStruct((NP, 1), jnp.float32)],
    )(s_parts, cnt_parts, *ins)


def _node_final_body(s2p, cnt, esump, x, pw, pb, gw, gb, g2, b2,
                     nw1, nb1, nw2, nb2, nw3, nb3, xc_out, np_out):
    c = jnp.maximum(cnt[...], 1.0)
    s2 = s2p[0] + s2p[1]
    x2 = jax.nn.relu(_ln(s2 / c, g2[...], b2[...]))
    xv = x[...]
    skip = xv @ pw[...] + pb[...]
    gate = jax.nn.sigmoid(skip @ gw[...] + gb[...])
    xf = gate * skip + (1.0 - gate) * x2
    esum = esump[0] + esump[1]
    efm = esum / c
    xc = jnp.concatenate([xf, efm], axis=1)

    def _elu(v):
        return jnp.where(v > 0, v, jnp.exp(jnp.minimum(v, 0.0)) - 1.0)

    h2 = _elu(xc @ nw1[...] + nb1[...])
    h2 = _elu(h2 @ nw2[...] + nb2[...])
    np_out[...] = h2 @ nw3[...] + nb3[...]
    xc_out[...] = xc


def _node_final(s2_parts, cnt, esum_parts, x, p):
    ins = (p['proj_W'], p['proj_b'], p['gate_W'], p['gate_b'],
           p['bn2_g'], p['bn2_b'], p['np_W1'], p['np_b1'],
           p['np_W2'], p['np_b2'], p['np_W3'], p['np_b3'])
    return pl.pallas_call(
        _node_final_body,
        grid=(NP // NB,),
        in_specs=[pl.BlockSpec((NC, NB, 128), lambda i: (0, i, 0)),
                  _blk((NB, 1)),
                  pl.BlockSpec((NC, NB, 128), lambda i: (0, i, 0)),
                  _blk((NB, 128))] + [_full(a) for a in ins],
        out_specs=[_blk((NB, 256)), _blk((NB, 1))],
        out_shape=[jax.ShapeDtypeStruct((NP, 256), jnp.float32),
                   jax.ShapeDtypeStruct((NP, 1), jnp.float32)],
    )(s2_parts, cnt, esum_parts, x, *ins)


def _sc_gather_add(pd, ps, dst_c, src_c, split):
    d_out = 64 if split else 128
    mesh = plsc.VectorSubcoreMesh(core_axis_name="c", subcore_axis_name="s")

    @functools.partial(
        pl.kernel, mesh=mesh,
        out_type=jax.ShapeDtypeStruct((NCHUNK, CH, 128), jnp.float32),
        scratch_types=[
            pltpu.VMEM((1, CH), jnp.int32),
            pltpu.VMEM((1, CH), jnp.int32),
            pltpu.VMEM((CH, 128), jnp.float32),
            pltpu.VMEM((CH, 128), jnp.float32),
            pltpu.VMEM((CH, 128), jnp.float32),
            pltpu.SemaphoreType.DMA,
            pltpu.SemaphoreType.DMA,
        ])
    def k(pd_hbm, ps_hbm, dst_hbm, src_hbm, out_hbm,
          idxd, idxs, rowsd, rowss, obuf, semd, sems):
        wid = lax.axis_index("s") * NC + lax.axis_index("c")

        def body(j, carry):
            chunk = j * NW + wid

            @pl.when(chunk < NCHUNK)
            def _():
                cd = pltpu.async_copy(dst_hbm.at[chunk], idxd.at[0], semd)
                cs = pltpu.async_copy(src_hbm.at[chunk], idxs.at[0], sems)
                cd.wait()
                cs.wait()
                gd = pltpu.async_copy(pd_hbm.at[idxd.at[0]], rowsd, semd)
                gs = pltpu.async_copy(ps_hbm.at[idxs.at[0]], rowss, sems)
                gd.wait()
                gs.wait()

                def addrow(r, c2):
                    for kk in range(d_out // 16):
                        sl = pl.ds(kk * 16, 16)
                        if split:
                            sr = pl.ds(64 + kk * 16, 16)
                            obuf[r, sl] = rowsd[r, sl] + rowss[r, sr]
                        else:
                            obuf[r, sl] = rowsd[r, sl] + rowss[r, sl]
                    return c2

                lax.fori_loop(0, CH, addrow, 0)
                pltpu.sync_copy(obuf, out_hbm.at[chunk])

            return carry

        lax.fori_loop(0, JMAX, body, 0)

    out = k(pd, ps, dst_c, src_c)
    return out.reshape(E, 128)


def _sc_scatter_add(vals, dst_c):
    mesh = plsc.VectorSubcoreMesh(core_axis_name="c", subcore_axis_name="s")

    @functools.partial(
        pl.kernel, mesh=mesh,
        out_type=jax.ShapeDtypeStruct((NC, NP, 128), jnp.float32),
        scratch_types=[
            pltpu.VMEM((1, CH), jnp.int32),
            pltpu.VMEM((CH, 128), jnp.float32),
            pltpu.VMEM_SHARED((NP, 128), jnp.float32),
            pltpu.SemaphoreType.DMA,
            pltpu.SemaphoreType.DMA,
        ])
    def k(vals_hbm, dst_hbm, out_hbm, idx, vbuf, acc, sem1, sem2):
        cid = lax.axis_index("c")
        sid = lax.axis_index("s")
        wid = sid * NC + cid

        def zrow(r, c2):
            for kk in range(8):
                vbuf[r, pl.ds(kk * 16, 16)] = jnp.zeros((16,), jnp.float32)
            return c2

        lax.fori_loop(0, CH, zrow, 0)
        for t in range(RPT // CH):
            pltpu.sync_copy(vbuf, acc.at[pl.ds(sid * RPT + t * CH, CH)])
        plsc.subcore_barrier()

        def body(j, carry):
            chunk = j * NW + wid

            @pl.when(chunk < NCHUNK)
            def _():
                ci = pltpu.async_copy(dst_hbm.at[chunk], idx.at[0], sem1)
                cv = pltpu.async_copy(vals_hbm.at[chunk], vbuf, sem2)
                ci.wait()
                cv.wait()
                pltpu.sync_copy(vbuf, acc.at[idx.at[0]], add=True)

            return carry

        lax.fori_loop(0, JMAX, body, 0)
        plsc.subcore_barrier()

        sl = pl.ds(sid * RPT, RPT)
        pltpu.sync_copy(acc.at[sl], out_hbm.at[cid, sl])

    return k(vals.reshape(NCHUNK, CH, 128), dst_c)


def _sc_count(dst_c):
    mesh = plsc.VectorSubcoreMesh(core_axis_name="c", subcore_axis_name="s")

    @functools.partial(
        pl.kernel, mesh=mesh,
        out_type=jax.ShapeDtypeStruct((NC, NP, 128), jnp.float32),
        scratch_types=[
            pltpu.VMEM((1, CH), jnp.int32),
            pltpu.VMEM((CH, 128), jnp.float32),
            pltpu.VMEM_SHARED((NP, 128), jnp.float32),
            pltpu.SemaphoreType.DMA,
        ])
    def k(dst_hbm, out_hbm, idx, ones, acc, sem1):
        cid = lax.axis_index("c")
        sid = lax.axis_index("s")
        wid = sid * NC + cid

        def zrow(r, c2):
            for kk in range(8):
                ones[r, pl.ds(kk * 16, 16)] = jnp.zeros((16,), jnp.float32)
            return c2

        lax.fori_loop(0, CH, zrow, 0)
        for t in range(RPT // CH):
            pltpu.sync_copy(ones, acc.at[pl.ds(sid * RPT + t * CH, CH)])

        def orow(r, c2):
            for kk in range(8):
                ones[r, pl.ds(kk * 16, 16)] = jnp.ones((16,), jnp.float32)
            return c2

        lax.fori_loop(0, CH, orow, 0)
        plsc.subcore_barrier()

        def body(j, carry):
            chunk = j * NW + wid

            @pl.when(chunk < NCHUNK)
            def _():
                pltpu.async_copy(dst_hbm.at[chunk], idx.at[0], sem1).wait()
                pltpu.sync_copy(ones, acc.at[idx.at[0]], add=True)

            return carry

        lax.fori_loop(0, JMAX, body, 0)
        plsc.subcore_barrier()

        sl = pl.ds(sid * RPT, RPT)
        pltpu.sync_copy(acc.at[sl], out_hbm.at[cid, sl])

    return k(dst_c)


def kernel(x_in, edge_attr, params, edge_index):
    p = params
    x0 = x_in[0]
    ea = edge_attr[0]
    src = edge_index[0, 0]
    dst = edge_index[0, 1]
    dst_c = dst.reshape(NCHUNK, CH)
    src_c = src.reshape(NCHUNK, CH)
    x_pad = jnp.pad(x0, ((0, NP - N), (0, 0)))

    x, p1 = _node_prep(x_pad, p)
    e_enc = _edge_enc(ea, p)
    esum_parts = _sc_scatter_add(e_enc, dst_c)
    cnt_parts = _sc_count(dst_c)

    pre1 = _sc_gather_add(p1, p1, dst_c, src_c, split=True)
    w3p = jnp.pad(p['c1_W3'], ((0, 0), (0, 64)))
    b3p = jnp.pad(p['c1_b3'], (0, 64))
    m1 = _conv_mlp(pre1, e_enc, 64, p['c1_W1'][2 * INDIM:], p['c1_b1'],
                   p['c1_W2'], p['c1_b2'], w3p, b3p)
    s1_parts = _sc_scatter_add(m1, dst_c)

    p2d, p2s, cnt = _node_mid(s1_parts, cnt_parts, p)
    pre2 = _sc_gather_add(p2d, p2s, dst_c, src_c, split=False)
    m2 = _conv_mlp(pre2, e_enc, 128, p['c2_W1'][128:], p['c2_b1'],
                   p['c2_W2'], p['c2_b2'], p['c2_W3'], p['c2_b3'])
    s2_parts = _sc_scatter_add(m2, dst_c)

    xc, node_probs = _node_final(s2_parts, cnt, esum_parts, x, p)
    return (xc[:N][None], node_probs[:N][None])


# speedup vs baseline: 4.3974x; 1.1959x over previous
import functools

import jax
import jax.numpy as jnp
from jax import lax
from jax.experimental import pallas as pl
from jax.experimental.pallas import tpu as pltpu
from jax.experimental.pallas import tpu_sc as plsc

N = 10000
NP = 10240
E = 320000
INDIM = 128
EDGE_DIM = 16

CH = 128
NCHUNK = E // CH
NC = 2
NS = 16
NW = NC * NS
JMAX = 80
NCHUNK_PAD = NW * JMAX
RPT = NP // NS

EB = 2560
NB = 1280


def _ln(x, g, b, eps=1e-6):
    mu = jnp.mean(x, axis=-1, keepdims=True)
    var = jnp.mean((x - mu) ** 2, axis=-1, keepdims=True)
    return (x - mu) / jnp.sqrt(var + eps) * g + b


def _full(a):
    return pl.BlockSpec(a.shape, lambda i: (0,) * a.ndim)


def _blk(shape):
    nd = len(shape)
    return pl.BlockSpec(shape, lambda i: (i,) + (0,) * (nd - 1))


def _edge_enc_body(ea, g, b, w1, b1, w2, b2, w3, b3, cw1, cb1, cw2, cb2, out):
    eav = ea[...]
    e = _ln(eav, g[...], b[...])
    e = jax.nn.relu(e @ w1[...] + b1[...])
    e = jax.nn.relu(e @ w2[...] + b2[...])
    e = e @ w3[...] + b3[...]
    h = jax.nn.relu(eav @ cw1[...] + cb1[...])
    ew = jax.nn.sigmoid(h @ cw2[...] + cb2[...])
    out[...] = e * ew


def _edge_enc(e_attr, p):
    ins = (p['ee_ln_g'], p['ee_ln_b'], p['ee_W1'], p['ee_b1'], p['ee_W2'],
           p['ee_b2'], p['ee_W3'], p['ee_b3'], p['ec_W1'], p['ec_b1'],
           p['ec_W2'], p['ec_b2'])
    return pl.pallas_call(
        _edge_enc_body,
        grid=(E // EB,),
        in_specs=[_blk((EB, EDGE_DIM))] + [_full(a) for a in ins],
        out_specs=_blk((EB, 128)),
        out_shape=jax.ShapeDtypeStruct((E, 128), jnp.float32),
    )(e_attr, *ins)


def _node_prep_body(xin, dummy, g, b, w1ds, x_out, p1_out):
    xv = xin[...]
    inv = xv[:, :1] == -999.0
    xv = jnp.where(inv, dummy[...][None, :], xv)
    xv = _ln(xv, g[...], b[...])
    x_out[...] = xv
    p1_out[...] = xv @ w1ds[...]


def _node_prep(x_pad, p):
    w1ds = jnp.concatenate([p['c1_W1'][:INDIM], p['c1_W1'][INDIM:2 * INDIM]],
                           axis=1)
    ins = (p['dummy'], p['bn0_g'], p['bn0_b'], w1ds)
    return pl.pallas_call(
        _node_prep_body,
        grid=(NP // NB,),
        in_specs=[_blk((NB, INDIM))] + [_full(a) for a in ins],
        out_specs=[_blk((NB, INDIM)), _blk((NB, 128))],
        out_shape=[jax.ShapeDtypeStruct((NP, INDIM), jnp.float32),
                   jax.ShapeDtypeStruct((NP, 128), jnp.float32)],
    )(x_pad, *ins)


def _conv_mlp(pre, e_enc, pre_cols, w1e, b1, w2, b2, w3, b3):
    def body(pre_r, e_r, w1e_r, b1_r, w2_r, b2_r, w3_r, b3_r, out):
        h = jax.nn.relu(pre_r[...][:, :pre_cols] + e_r[...] @ w1e_r[...]
                        + b1_r[...])
        h = jax.nn.relu(h @ w2_r[...] + b2_r[...])
        out[...] = h @ w3_r[...] + b3_r[...]

    ins = (w1e, b1, w2, b2, w3, b3)
    return pl.pallas_call(
        body,
        grid=(E // EB,),
        in_specs=[_blk((EB, 128)), _blk((EB, 128))] + [_full(a) for a in ins],
        out_specs=_blk((EB, 128)),
        out_shape=jax.ShapeDtypeStruct((E, 128), jnp.float32),
    )(pre, e_enc, *ins)


def _node_mid_body(sp, cntp, g, b, w2d, w2s, pd_out, ps_out, cnt_out):
    cv = cntp[...]
    cnt = cv[0, :, 0] + cv[1, :, 0]
    sv = sp[...][:, :, :64]
    s = sv[0] + sv[1]
    x1 = s / jnp.maximum(cnt, 1.0)[:, None]
    x1 = _ln(x1, g[...], b[...])
    x1 = jnp.where(x1 >= 0, x1, 0.01 * x1)
    pd_out[...] = x1 @ w2d[...]
    ps_out[...] = x1 @ w2s[...]
    cnt_out[...] = cnt[:, None]


def _node_mid(s_parts, cnt_parts, p):
    w2d = p['c2_W1'][:64]
    w2s = p['c2_W1'][64:128]
    ins = (p['bn1_g'], p['bn1_b'], w2d, w2s)
    return pl.pallas_call(
        _node_mid_body,
        grid=(NP // NB,),
        in_specs=[pl.BlockSpec((NC, NB, 128), lambda i: (0, i, 0)),
                  pl.BlockSpec((NC, NB, 128), lambda i: (0, i, 0))]
                 + [_full(a) for a in ins],
        out_specs=[_blk((NB, 128)), _blk((NB, 128)), _blk((NB, 1))],
        out_shape=[jax.ShapeDtypeStruct((NP, 128), jnp.float32),
                   jax.ShapeDtypeStruct((NP, 128), jnp.float32),
                   jax.ShapeDtypeStruct((NP, 1), jnp.float32)],
    )(s_parts, cnt_parts, *ins)


def _node_final_body(s2p, cnt, esump, x, pw, pb, gw, gb, g2, b2,
                     nw1, nb1, nw2, nb2, nw3, nb3, xc_out, np_out):
    c = jnp.maximum(cnt[...], 1.0)
    s2 = s2p[0] + s2p[1]
    x2 = jax.nn.relu(_ln(s2 / c, g2[...], b2[...]))
    xv = x[...]
    skip = xv @ pw[...] + pb[...]
    gate = jax.nn.sigmoid(skip @ gw[...] + gb[...])
    xf = gate * skip + (1.0 - gate) * x2
    esum = esump[0] + esump[1]
    efm = esum / c
    xc = jnp.concatenate([xf, efm], axis=1)

    def _elu(v):
        return jnp.where(v > 0, v, jnp.exp(jnp.minimum(v, 0.0)) - 1.0)

    h2 = _elu(xc @ nw1[...] + nb1[...])
    h2 = _elu(h2 @ nw2[...] + nb2[...])
    np_out[...] = h2 @ nw3[...] + nb3[...]
    xc_out[...] = xc


def _node_final(s2_parts, cnt, esum_parts, x, p):
    ins = (p['proj_W'], p['proj_b'], p['gate_W'], p['gate_b'],
           p['bn2_g'], p['bn2_b'], p['np_W1'], p['np_b1'],
           p['np_W2'], p['np_b2'], p['np_W3'], p['np_b3'])
    return pl.pallas_call(
        _node_final_body,
        grid=(NP // NB,),
        in_specs=[pl.BlockSpec((NC, NB, 128), lambda i: (0, i, 0)),
                  _blk((NB, 1)),
                  pl.BlockSpec((NC, NB, 128), lambda i: (0, i, 0)),
                  _blk((NB, 128))] + [_full(a) for a in ins],
        out_specs=[_blk((NB, 256)), _blk((NB, 1))],
        out_shape=[jax.ShapeDtypeStruct((NP, 256), jnp.float32),
                   jax.ShapeDtypeStruct((NP, 1), jnp.float32)],
    )(s2_parts, cnt, esum_parts, x, *ins)


def _sc_gather_add(pd, ps, dst_c, src_c, split):
    d_out = 64 if split else 128
    mesh = plsc.VectorSubcoreMesh(core_axis_name="c", subcore_axis_name="s")

    @functools.partial(
        pl.kernel, mesh=mesh,
        out_type=jax.ShapeDtypeStruct((NCHUNK, CH, 128), jnp.float32),
        scratch_types=[
            pltpu.VMEM((JMAX, CH), jnp.int32),
            pltpu.VMEM((JMAX, CH), jnp.int32),
            pltpu.VMEM((2, CH, 128), jnp.float32),
            pltpu.VMEM((2, CH, 128), jnp.float32),
            pltpu.SemaphoreType.DMA,
            pltpu.SemaphoreType.DMA,
            pltpu.SemaphoreType.DMA,
            pltpu.SemaphoreType.DMA,
            pltpu.SemaphoreType.DMA,
            pltpu.SemaphoreType.DMA,
        ])
    def k(pd_hbm, ps_hbm, dst_hbm, src_hbm, out_hbm,
          idxd, idxs, rowsd, rowss, sd0, sd1, ss0, ss1, st0, st1):
        wid = lax.axis_index("s") * NC + lax.axis_index("c")
        base = wid * JMAX
        nvalid = jnp.minimum(JMAX, NCHUNK - base)
        pltpu.sync_copy(dst_hbm.at[pl.ds(base, JMAX)], idxd)
        pltpu.sync_copy(src_hbm.at[pl.ds(base, JMAX)], idxs)
        sd = (sd0, sd1)
        ss = (ss0, ss1)
        st = (st0, st1)

        def fire(j, b):
            @pl.when(j < nvalid)
            def _():
                pltpu.async_copy(pd_hbm.at[idxd.at[j]], rowsd.at[b], sd[b])
                pltpu.async_copy(ps_hbm.at[idxs.at[j]], rowss.at[b], ss[b])

        def wait_gather(j, b):
            pltpu.make_async_copy(pd_hbm.at[idxd.at[j]], rowsd.at[b],
                                  sd[b]).wait()
            pltpu.make_async_copy(ps_hbm.at[idxs.at[j]], rowss.at[b],
                                  ss[b]).wait()

        def wait_store(b):
            pltpu.make_async_copy(rowsd.at[b], out_hbm.at[base],
                                  st[b]).wait()

        fire(0, 0)

        def step(t, carry):
            for b in range(2):
                j = 2 * t + b
                b1 = 1 - b

                @pl.when(j < nvalid)
                def _():
                    wait_gather(j, b)
                    @pl.when(j >= 1)
                    def _():
                        wait_store(b1)

                    fire(j + 1, b1)

                    def addrow(r, c2):
                        for kk in range(d_out // 16):
                            sl = pl.ds(kk * 16, 16)
                            if split:
                                sr = pl.ds(64 + kk * 16, 16)
                                rowsd[b, r, sl] = (rowsd[b, r, sl]
                                                   + rowss[b, r, sr])
                            else:
                                rowsd[b, r, sl] = (rowsd[b, r, sl]
                                                   + rowss[b, r, sl])
                        return c2

                    lax.fori_loop(0, CH, addrow, 0)
                    pltpu.async_copy(rowsd.at[b], out_hbm.at[base + j],
                                     st[b])

            return carry

        lax.fori_loop(0, (JMAX + 1) // 2, step, 0)
        par = lax.rem(nvalid - 1, 2)
        for b in range(2):
            @pl.when(par == b)
            def _():
                wait_store(b)

    out = k(pd, ps, dst_c, src_c)
    return out.reshape(E, 128)


def _sc_scatter_add(vals, dst_c):
    mesh = plsc.VectorSubcoreMesh(core_axis_name="c", subcore_axis_name="s")

    @functools.partial(
        pl.kernel, mesh=mesh,
        out_type=jax.ShapeDtypeStruct((NC, NP, 128), jnp.float32),
        scratch_types=[
            pltpu.VMEM((JMAX, CH), jnp.int32),
            pltpu.VMEM((2, CH, 128), jnp.float32),
            pltpu.VMEM_SHARED((NP, 128), jnp.float32),
            pltpu.SemaphoreType.DMA,
            pltpu.SemaphoreType.DMA,
        ])
    def k(vals_hbm, dst_hbm, out_hbm, idx, vbuf, acc, sl0, sl1):
        cid = lax.axis_index("c")
        sid = lax.axis_index("s")
        wid = sid * NC + cid
        base = wid * JMAX
        nvalid = jnp.minimum(JMAX, NCHUNK - base)
        slm = (sl0, sl1)

        def zrow(r, c2):
            for kk in range(8):
                vbuf[0, r, pl.ds(kk * 16, 16)] = jnp.zeros((16,), jnp.float32)
            return c2

        lax.fori_loop(0, CH, zrow, 0)
        for t in range(RPT // CH):
            pltpu.sync_copy(vbuf.at[0], acc.at[pl.ds(sid * RPT + t * CH, CH)])
        pltpu.sync_copy(dst_hbm.at[pl.ds(base, JMAX)], idx)
        plsc.subcore_barrier()

        def fire(j, b):
            @pl.when(j < nvalid)
            def _():
                pltpu.async_copy(vals_hbm.at[base + j], vbuf.at[b], slm[b])

        fire(0, 0)

        def step(t, carry):
            for b in range(2):
                j = 2 * t + b

                @pl.when(j < nvalid)
                def _():
                    pltpu.make_async_copy(vals_hbm.at[base + j], vbuf.at[b],
                                          slm[b]).wait()
                    fire(j + 1, 1 - b)
                    pltpu.sync_copy(vbuf.at[b], acc.at[idx.at[j]], add=True)

            return carry

        lax.fori_loop(0, (JMAX + 1) // 2, step, 0)
        plsc.subcore_barrier()

        sl = pl.ds(sid * RPT, RPT)
        pltpu.sync_copy(acc.at[sl], out_hbm.at[cid, sl])

    return k(vals.reshape(NCHUNK, CH, 128), dst_c)


def _sc_count(dst_c):
    mesh = plsc.VectorSubcoreMesh(core_axis_name="c", subcore_axis_name="s")

    @functools.partial(
        pl.kernel, mesh=mesh,
        out_type=jax.ShapeDtypeStruct((NC, NP, 128), jnp.float32),
        scratch_types=[
            pltpu.VMEM((JMAX, CH), jnp.int32),
            pltpu.VMEM((CH, 128), jnp.float32),
            pltpu.VMEM_SHARED((NP, 128), jnp.float32),
            pltpu.SemaphoreType.DMA,
        ])
    def k(dst_hbm, out_hbm, idx, ones, acc, sem1):
        cid = lax.axis_index("c")
        sid = lax.axis_index("s")
        wid = sid * NC + cid
        base = wid * JMAX
        nvalid = jnp.minimum(JMAX, NCHUNK - base)

        def zrow(r, c2):
            for kk in range(8):
                ones[r, pl.ds(kk * 16, 16)] = jnp.zeros((16,), jnp.float32)
            return c2

        lax.fori_loop(0, CH, zrow, 0)
        for t in range(RPT // CH):
            pltpu.sync_copy(ones, acc.at[pl.ds(sid * RPT + t * CH, CH)])

        def orow(r, c2):
            for kk in range(8):
                ones[r, pl.ds(kk * 16, 16)] = jnp.ones((16,), jnp.float32)
            return c2

        lax.fori_loop(0, CH, orow, 0)
        pltpu.sync_copy(dst_hbm.at[pl.ds(base, JMAX)], idx)
        plsc.subcore_barrier()

        def body(j, carry):
            @pl.when(j < nvalid)
            def _():
                pltpu.sync_copy(ones, acc.at[idx.at[j]], add=True)

            return carry

        lax.fori_loop(0, JMAX, body, 0)
        plsc.subcore_barrier()

        sl = pl.ds(sid * RPT, RPT)
        pltpu.sync_copy(acc.at[sl], out_hbm.at[cid, sl])

    return k(dst_c)


def kernel(x_in, edge_attr, params, edge_index):
    p = params
    x0 = x_in[0]
    ea = edge_attr[0]
    src = edge_index[0, 0]
    dst = edge_index[0, 1]
    pad_rows = ((0, NCHUNK_PAD - NCHUNK), (0, 0))
    dst_c = jnp.pad(dst.reshape(NCHUNK, CH), pad_rows)
    src_c = jnp.pad(src.reshape(NCHUNK, CH), pad_rows)
    x_pad = jnp.pad(x0, ((0, NP - N), (0, 0)))

    x, p1 = _node_prep(x_pad, p)
    e_enc = _edge_enc(ea, p)
    esum_parts = _sc_scatter_add(e_enc, dst_c)
    cnt_parts = _sc_count(dst_c)

    pre1 = _sc_gather_add(p1, p1, dst_c, src_c, split=True)
    w3p = jnp.pad(p['c1_W3'], ((0, 0), (0, 64)))
    b3p = jnp.pad(p['c1_b3'], (0, 64))
    m1 = _conv_mlp(pre1, e_enc, 64, p['c1_W1'][2 * INDIM:], p['c1_b1'],
                   p['c1_W2'], p['c1_b2'], w3p, b3p)
    s1_parts = _sc_scatter_add(m1, dst_c)

    p2d, p2s, cnt = _node_mid(s1_parts, cnt_parts, p)
    pre2 = _sc_gather_add(p2d, p2s, dst_c, src_c, split=False)
    m2 = _conv_mlp(pre2, e_enc, 128, p['c2_W1'][128:], p['c2_b1'],
                   p['c2_W2'], p['c2_b2'], p['c2_W3'], p['c2_b3'])
    s2_parts = _sc_scatter_add(m2, dst_c)

    xc, node_probs = _node_final(s2_parts, cnt, esum_parts, x, p)
    return (xc[:N][None], node_probs[:N][None])
